# baseline (speedup 1.0000x reference)
import functools

import jax
import jax.numpy as jnp
from jax import lax
from jax.experimental import pallas as pl
from jax.experimental.pallas import tpu as pltpu
from jax.experimental.pallas import tpu_sc as plsc

NC, NS = 2, 16
NW = NC * NS
C0, C1 = 96, 104


@functools.partial(jax.jit, static_argnames=("b", "l", "e"))
def _sc_lookup(de1, de2, f3d, t1, t2, *, b, l, e):
    rows_per_w = b // NW
    mesh = plsc.VectorSubcoreMesh(core_axis_name="c", subcore_axis_name="s")

    @functools.partial(
        pl.kernel,
        out_type=jax.ShapeDtypeStruct((b, l, 2 * e + 1), jnp.float32),
        mesh=mesh,
        scratch_types=[
            pltpu.VMEM((l,), jnp.int32),
            pltpu.VMEM((l,), jnp.int32),
            pltpu.VMEM((C0, e), jnp.float32),
            pltpu.VMEM((C1, e), jnp.float32),
            pltpu.VMEM((C0, e), jnp.float32),
            pltpu.VMEM((C1, e), jnp.float32),
            pltpu.SemaphoreType.DMA,
            pltpu.SemaphoreType.DMA,
            pltpu.SemaphoreType.DMA,
            pltpu.SemaphoreType.DMA,
            pltpu.SemaphoreType.DMA,
        ],
        compiler_params=pltpu.CompilerParams(use_tc_tiling_on_sc=False),
    )
    def k(de1_hbm, de2_hbm, f_hbm, t1_hbm, t2_hbm, out_hbm,
          idx1_v, idx2_v, r1a_v, r1b_v, r2a_v, r2b_v,
          s1, s2, s3, s4, sf):
        wid = lax.axis_index("s") * NC + lax.axis_index("c")
        row0 = wid * rows_per_w

        def body(i, carry):
            row = row0 + i
            pltpu.sync_copy(de1_hbm.at[row], idx1_v)
            pltpu.sync_copy(de2_hbm.at[row], idx2_v)
            cpf = pltpu.async_copy(
                f_hbm.at[row], out_hbm.at[row, :, pl.ds(2 * e, 1)], sf)
            g1 = pltpu.async_copy(t1_hbm.at[idx1_v.at[pl.ds(0, C0)]], r1a_v, s1)
            g2 = pltpu.async_copy(t1_hbm.at[idx1_v.at[pl.ds(C0, C1)]], r1b_v, s2)
            g3 = pltpu.async_copy(t2_hbm.at[idx2_v.at[pl.ds(0, C0)]], r2a_v, s3)
            g4 = pltpu.async_copy(t2_hbm.at[idx2_v.at[pl.ds(C0, C1)]], r2b_v, s4)
            g1.wait()
            pltpu.sync_copy(r1a_v, out_hbm.at[row, pl.ds(0, C0), pl.ds(0, e)])
            g2.wait()
            pltpu.sync_copy(r1b_v, out_hbm.at[row, pl.ds(C0, C1), pl.ds(0, e)])
            g3.wait()
            pltpu.sync_copy(r2a_v, out_hbm.at[row, pl.ds(0, C0), pl.ds(e, e)])
            g4.wait()
            pltpu.sync_copy(r2b_v, out_hbm.at[row, pl.ds(C0, C1), pl.ds(e, e)])
            cpf.wait()
            return carry

        lax.fori_loop(0, rows_per_w, body, 0)

    return k(de1, de2, f3d, t1, t2)


def kernel(de1, de2, f, de1_table, de2_table):
    b, l = de1.shape
    _, e = de1_table.shape
    return _sc_lookup(de1, de2, f.reshape(b, l, 1), de1_table, de2_table,
                      b=b, l=l, e=e)


# speedup vs baseline: 5.8376x; 5.8376x over previous
import functools

import jax
import jax.numpy as jnp
from jax import lax
from jax.experimental import pallas as pl
from jax.experimental.pallas import tpu as pltpu
from jax.experimental.pallas import tpu_sc as plsc

NC, NS = 2, 16
NW = NC * NS
KP = 4
LP = 40


@functools.partial(jax.jit, static_argnames=("b", "l", "e", "v"))
def _sc_lookup_t(de1t, de2t, ft, t1f, t2f, *, b, l, e, v):
    bstr = b // NW
    n_piece = l // LP
    n_pass = e // KP
    mesh = plsc.VectorSubcoreMesh(core_axis_name="c", subcore_axis_name="s")

    @functools.partial(
        pl.kernel,
        out_type=jax.ShapeDtypeStruct((2 * e + 1, l, b), jnp.float32),
        mesh=mesh,
        scratch_types=[
            pltpu.VMEM((l, bstr), jnp.int32),
            pltpu.VMEM((l, bstr), jnp.int32),
            pltpu.VMEM((KP * v,), jnp.float32),
            pltpu.VMEM((LP, bstr), jnp.float32),
            pltpu.VMEM((LP, bstr), jnp.float32),
            pltpu.VMEM((LP, bstr), jnp.float32),
            pltpu.VMEM((LP, bstr), jnp.float32),
            pltpu.VMEM((LP, bstr), jnp.float32),
            pltpu.VMEM((LP, bstr), jnp.float32),
            pltpu.VMEM((LP, bstr), jnp.float32),
            pltpu.VMEM((LP, bstr), jnp.float32),
            pltpu.SemaphoreType.DMA,
            pltpu.SemaphoreType.DMA,
        ],
        compiler_params=pltpu.CompilerParams(
            use_tc_tiling_on_sc=True, needs_layout_passes=False),
    )
    def k(de1_hbm, de2_hbm, f_hbm, t1_hbm, t2_hbm, out_hbm,
          i1_v, i2_v, rows_v,
          b00, b01, b02, b03, b10, b11, b12, b13, sem0, sem1):
        bufs = ((b00, b01, b02, b03), (b10, b11, b12, b13))
        sems = (sem0, sem1)
        w = lax.axis_index("s") * NC + lax.axis_index("c")
        b0 = w * bstr

        pltpu.sync_copy(de1_hbm.at[:, pl.ds(b0, bstr)], i1_v)
        pltpu.sync_copy(de2_hbm.at[:, pl.ds(b0, bstr)], i2_v)

        for piece in range(n_piece):
            l0 = piece * LP
            pltpu.sync_copy(f_hbm.at[pl.ds(l0, LP), pl.ds(b0, bstr)], b00)
            pltpu.sync_copy(b00, out_hbm.at[2 * e, pl.ds(l0, LP), pl.ds(b0, bstr)])

        for tbl_i, (tf_hbm, i_v, cbase) in enumerate(
                ((t1_hbm, i1_v, 0), (t2_hbm, i2_v, e))):

            def pass_body(p, carry, tf_hbm=tf_hbm, i_v=i_v, cbase=cbase,
                          tbl_i=tbl_i):
                pltpu.sync_copy(tf_hbm.at[pl.ds(p * (KP * v), KP * v)], rows_v)
                for piece in range(n_piece):
                    s = piece % 2
                    l0 = piece * LP

                    def drain(s=s, l0=l0):
                        for r in range(KP):
                            pltpu.make_async_copy(
                                bufs[s][r],
                                out_hbm.at[cbase, pl.ds(l0, LP), pl.ds(b0, bstr)],
                                sems[s]).wait()

                    if tbl_i > 0 or piece >= 2:
                        drain()
                    else:
                        pl.when(p > 0)(drain)

                    def fill(ll, carry2, i_v=i_v, s=s, l0=l0):
                        lrow = l0 + ll
                        for j in range(bstr // 16):
                            idx = i_v[lrow, pl.ds(j * 16, 16)]
                            for r in range(KP):
                                vals = plsc.load_gather(
                                    rows_v, [idx + r * v if r else idx])
                                bufs[s][r][ll, pl.ds(j * 16, 16)] = vals
                        return carry2

                    lax.fori_loop(0, LP, fill, 0)
                    for r in range(KP):
                        c = cbase + p * KP + r
                        pltpu.async_copy(
                            bufs[s][r],
                            out_hbm.at[c, pl.ds(l0, LP), pl.ds(b0, bstr)],
                            sems[s])
                return carry

            lax.fori_loop(0, n_pass, pass_body, 0)

        for s in range(2):
            for r in range(KP):
                pltpu.make_async_copy(
                    bufs[s][r],
                    out_hbm.at[0, pl.ds(0, LP), pl.ds(b0, bstr)],
                    sems[s]).wait()

    return k(de1t, de2t, ft, t1f, t2f)


def kernel(de1, de2, f, de1_table, de2_table):
    b, l = de1.shape
    v, e = de1_table.shape
    out_t = _sc_lookup_t(
        de1.T, de2.T, f.T,
        de1_table.T.reshape(-1), de2_table.T.reshape(-1),
        b=b, l=l, e=e, v=v)
    return out_t.transpose(2, 1, 0)


# speedup vs baseline: 15.4840x; 2.6525x over previous
import functools

import jax
import jax.numpy as jnp
from jax import lax
from jax.experimental import pallas as pl
from jax.experimental.pallas import tpu as pltpu
from jax.experimental.pallas import tpu_sc as plsc

NC, NS = 2, 16
NW = NC * NS
KP = 4
LP = 40


@functools.partial(jax.jit, static_argnames=("b", "l", "e", "v"))
def _sc_lookup_t(de1t, de2t, ft, t1f, t2f, *, b, l, e, v):
    bstr = b // NW
    n_piece = l // LP
    n_pass = e // KP
    mesh = plsc.VectorSubcoreMesh(core_axis_name="c", subcore_axis_name="s")

    @functools.partial(
        pl.kernel,
        out_type=jax.ShapeDtypeStruct((2 * e + 1, l, b), jnp.float32),
        mesh=mesh,
        scratch_types=[
            pltpu.VMEM((l, bstr), jnp.int32),
            pltpu.VMEM((l, bstr), jnp.int32),
            pltpu.VMEM((KP * v,), jnp.float32),
            pltpu.VMEM((LP, bstr), jnp.float32),
            pltpu.VMEM((LP, bstr), jnp.float32),
            pltpu.VMEM((LP, bstr), jnp.float32),
            pltpu.VMEM((LP, bstr), jnp.float32),
            pltpu.VMEM((LP, bstr), jnp.float32),
            pltpu.VMEM((LP, bstr), jnp.float32),
            pltpu.VMEM((LP, bstr), jnp.float32),
            pltpu.VMEM((LP, bstr), jnp.float32),
            pltpu.SemaphoreType.DMA,
            pltpu.SemaphoreType.DMA,
        ],
        compiler_params=pltpu.CompilerParams(
            use_tc_tiling_on_sc=True, needs_layout_passes=False),
    )
    def k(de1_hbm, de2_hbm, f_hbm, t1_hbm, t2_hbm, out_hbm,
          i1_v, i2_v, rows_v,
          b00, b01, b02, b03, b10, b11, b12, b13, sem0, sem1):
        bufs = ((b00, b01, b02, b03), (b10, b11, b12, b13))
        sems = (sem0, sem1)
        w = lax.axis_index("s") * NC + lax.axis_index("c")
        b0 = w * bstr

        pltpu.sync_copy(de1_hbm.at[:, pl.ds(b0, bstr)], i1_v)
        pltpu.sync_copy(de2_hbm.at[:, pl.ds(b0, bstr)], i2_v)

        for piece in range(n_piece):
            l0 = piece * LP
            pltpu.sync_copy(f_hbm.at[pl.ds(l0, LP), pl.ds(b0, bstr)], b00)
            pltpu.sync_copy(b00, out_hbm.at[2 * e, pl.ds(l0, LP), pl.ds(b0, bstr)])

        for tbl_i, (tf_hbm, i_v, cbase) in enumerate(
                ((t1_hbm, i1_v, 0), (t2_hbm, i2_v, e))):

            def pass_body(p, carry, tf_hbm=tf_hbm, i_v=i_v, cbase=cbase,
                          tbl_i=tbl_i):
                pltpu.sync_copy(tf_hbm.at[pl.ds(p * (KP * v), KP * v)], rows_v)
                for piece in range(n_piece):
                    s = piece % 2
                    l0 = piece * LP

                    def drain(s=s, l0=l0):
                        for r in range(KP):
                            pltpu.make_async_copy(
                                bufs[s][r],
                                out_hbm.at[cbase, pl.ds(l0, LP), pl.ds(b0, bstr)],
                                sems[s]).wait()

                    if tbl_i > 0 or piece >= 2:
                        drain()
                    else:
                        pl.when(p > 0)(drain)

                    @plsc.parallel_loop(0, LP, 1, unroll=2)
                    def fill(ll, i_v=i_v, s=s, l0=l0):
                        lrow = l0 + ll
                        for j in range(bstr // 16):
                            idx = i_v[lrow, pl.ds(j * 16, 16)]
                            for r in range(KP):
                                vals = plsc.load_gather(
                                    rows_v, [idx + r * v if r else idx])
                                bufs[s][r][ll, pl.ds(j * 16, 16)] = vals
                    for r in range(KP):
                        c = cbase + p * KP + r
                        pltpu.async_copy(
                            bufs[s][r],
                            out_hbm.at[c, pl.ds(l0, LP), pl.ds(b0, bstr)],
                            sems[s])
                return carry

            lax.fori_loop(0, n_pass, pass_body, 0)

        for s in range(2):
            for r in range(KP):
                pltpu.make_async_copy(
                    bufs[s][r],
                    out_hbm.at[0, pl.ds(0, LP), pl.ds(b0, bstr)],
                    sems[s]).wait()

    return k(de1t, de2t, ft, t1f, t2f)


def kernel(de1, de2, f, de1_table, de2_table):
    b, l = de1.shape
    v, e = de1_table.shape
    out_t = _sc_lookup_t(
        de1.T, de2.T, f.T,
        de1_table.T.reshape(-1), de2_table.T.reshape(-1),
        b=b, l=l, e=e, v=v)
    return out_t.transpose(2, 1, 0)


# speedup vs baseline: 15.5377x; 1.0035x over previous
import functools

import jax
import jax.numpy as jnp
from jax import lax
from jax.experimental import pallas as pl
from jax.experimental.pallas import tpu as pltpu
from jax.experimental.pallas import tpu_sc as plsc

NC, NS = 2, 16
NW = NC * NS
KP = 4
LP = 40


@functools.partial(jax.jit, static_argnames=("b", "l", "e", "v"))
def _sc_lookup_t(de1t, de2t, ft, t1f, t2f, *, b, l, e, v):
    bstr = b // NW
    n_piece = l // LP
    n_pass = e // KP
    mesh = plsc.VectorSubcoreMesh(core_axis_name="c", subcore_axis_name="s")

    @functools.partial(
        pl.kernel,
        out_type=jax.ShapeDtypeStruct((2 * e + 1, l, b), jnp.float32),
        mesh=mesh,
        scratch_types=[
            pltpu.VMEM((l, bstr), jnp.int32),
            pltpu.VMEM((l, bstr), jnp.int32),
            pltpu.VMEM((KP * v,), jnp.float32),
            pltpu.VMEM((LP, bstr), jnp.float32),
            pltpu.VMEM((LP, bstr), jnp.float32),
            pltpu.VMEM((LP, bstr), jnp.float32),
            pltpu.VMEM((LP, bstr), jnp.float32),
            pltpu.VMEM((LP, bstr), jnp.float32),
            pltpu.VMEM((LP, bstr), jnp.float32),
            pltpu.VMEM((LP, bstr), jnp.float32),
            pltpu.VMEM((LP, bstr), jnp.float32),
            pltpu.SemaphoreType.DMA,
            pltpu.SemaphoreType.DMA,
        ],
        compiler_params=pltpu.CompilerParams(
            use_tc_tiling_on_sc=True, needs_layout_passes=False),
    )
    def k(de1_hbm, de2_hbm, f_hbm, t1_hbm, t2_hbm, out_hbm,
          i1_v, i2_v, rows_v,
          b00, b01, b02, b03, b10, b11, b12, b13, sem0, sem1):
        bufs = ((b00, b01, b02, b03), (b10, b11, b12, b13))
        sems = (sem0, sem1)
        w = lax.axis_index("s") * NC + lax.axis_index("c")
        b0 = w * bstr

        pltpu.sync_copy(de1_hbm.at[:, pl.ds(b0, bstr)], i1_v)
        pltpu.sync_copy(de2_hbm.at[:, pl.ds(b0, bstr)], i2_v)

        for piece in range(n_piece):
            l0 = piece * LP
            pltpu.sync_copy(f_hbm.at[pl.ds(l0, LP), pl.ds(b0, bstr)], b00)
            pltpu.sync_copy(b00, out_hbm.at[2 * e, pl.ds(l0, LP), pl.ds(b0, bstr)])

        for tbl_i, (tf_hbm, i_v, cbase) in enumerate(
                ((t1_hbm, i1_v, 0), (t2_hbm, i2_v, e))):

            def pass_body(p, carry, tf_hbm=tf_hbm, i_v=i_v, cbase=cbase,
                          tbl_i=tbl_i):
                pltpu.sync_copy(tf_hbm.at[pl.ds(p * (KP * v), KP * v)], rows_v)
                for piece in range(n_piece):
                    s = piece % 2
                    l0 = piece * LP

                    def drain(s=s, l0=l0):
                        for r in range(KP):
                            pltpu.make_async_copy(
                                bufs[s][r],
                                out_hbm.at[cbase, pl.ds(l0, LP), pl.ds(b0, bstr)],
                                sems[s]).wait()

                    if tbl_i > 0 or piece >= 2:
                        drain()
                    else:
                        pl.when(p > 0)(drain)

                    @plsc.parallel_loop(0, LP, 1, unroll=4)
                    def fill(ll, i_v=i_v, s=s, l0=l0):
                        lrow = l0 + ll
                        for j in range(bstr // 16):
                            idx = i_v[lrow, pl.ds(j * 16, 16)]
                            for r in range(KP):
                                vals = plsc.load_gather(
                                    rows_v, [idx + r * v if r else idx])
                                bufs[s][r][ll, pl.ds(j * 16, 16)] = vals
                    for r in range(KP):
                        c = cbase + p * KP + r
                        pltpu.async_copy(
                            bufs[s][r],
                            out_hbm.at[c, pl.ds(l0, LP), pl.ds(b0, bstr)],
                            sems[s])
                return carry

            lax.fori_loop(0, n_pass, pass_body, 0)

        for s in range(2):
            for r in range(KP):
                pltpu.make_async_copy(
                    bufs[s][r],
                    out_hbm.at[0, pl.ds(0, LP), pl.ds(b0, bstr)],
                    sems[s]).wait()

    return k(de1t, de2t, ft, t1f, t2f)


def kernel(de1, de2, f, de1_table, de2_table):
    b, l = de1.shape
    v, e = de1_table.shape
    out_t = _sc_lookup_t(
        de1.T, de2.T, f.T,
        de1_table.T.reshape(-1), de2_table.T.reshape(-1),
        b=b, l=l, e=e, v=v)
    return out_t.transpose(2, 1, 0)
